# baseline (speedup 1.0000x reference)
import functools
import jax
import jax.numpy as jnp
from jax.experimental import pallas as pl
from jax.experimental.pallas import tpu as pltpu

_TN = 256
_KC = 1024
_GRP = 128


def _vq_body(x_ref, tt_ref, o_ref, bv_ref, bi_ref, *, nk, grp):
    i = pl.program_id(0)
    j = pl.program_id(1)
    x = x_ref[...]
    tt = tt_ref[...]
    dots = jax.lax.dot_general(
        x, tt, (((1,), (0,)), ((), ())),
        preferred_element_type=jnp.float32)
    x2 = jnp.sum(x * x, axis=1, keepdims=True)
    y2 = jnp.sum(tt * tt, axis=0, keepdims=True)
    score = (x2 + y2) - 2.0 * dots
    m = jnp.min(score, axis=1, keepdims=True)
    iota = jax.lax.broadcasted_iota(jnp.int32, score.shape, 1)
    arg = jnp.min(jnp.where(score == m, iota, _KC),
                  axis=1, keepdims=True) + j * _KC

    @pl.when(j == 0)
    def _init():
        bv_ref[...] = m
        bi_ref[...] = arg

    @pl.when(j > 0)
    def _merge():
        take = m < bv_ref[...]
        bv_ref[...] = jnp.where(take, m, bv_ref[...])
        bi_ref[...] = jnp.where(take, arg, bi_ref[...])

    @pl.when(j == nk - 1)
    def _deposit():
        col = jax.lax.broadcasted_iota(jnp.int32, o_ref.shape, 1)
        o_ref[...] = jnp.where(col == i % grp, bi_ref[...], o_ref[...])


def kernel(embeddings, table):
    B, S, D = embeddings.shape
    K = table.shape[0]
    N = B * S
    nk = K // _KC
    grp = min(_GRP, N // _TN)
    x = embeddings.reshape(N, D)
    tt = table.T
    out = pl.pallas_call(
        functools.partial(_vq_body, nk=nk, grp=grp),
        grid=(N // _TN, nk),
        in_specs=[
            pl.BlockSpec((_TN, D), lambda i, j: (i, 0)),
            pl.BlockSpec((D, _KC), lambda i, j: (0, j)),
        ],
        out_specs=pl.BlockSpec((_TN, grp), lambda i, j: (0, i // grp)),
        out_shape=jax.ShapeDtypeStruct((_TN, N // _TN), jnp.int32),
        scratch_shapes=[
            pltpu.VMEM((_TN, 1), jnp.float32),
            pltpu.VMEM((_TN, 1), jnp.int32),
        ],
    )(x, tt)
    return out.T.reshape(B, S)


# speedup vs baseline: 2.1795x; 2.1795x over previous
import functools
import jax
import jax.numpy as jnp
from jax.experimental import pallas as pl
from jax.experimental.pallas import tpu as pltpu

_TN = 256
_KC = 1024
_GRP = 128


def _vq_body(x_ref, tt_ref, o_ref, y2_ref, *, nk, grp):
    i = pl.program_id(0)

    @pl.when(i == 0)
    def _norms():
        tt = tt_ref[...]
        y2_ref[...] = jnp.sum(tt * tt, axis=0, keepdims=True)

    x = x_ref[...]
    x2 = jnp.sum(x * x, axis=1, keepdims=True)

    best_v = None
    best_i = None
    for j in range(nk):
        tt = tt_ref[:, j * _KC:(j + 1) * _KC]
        dots = jax.lax.dot_general(
            x, tt, (((1,), (0,)), ((), ())),
            preferred_element_type=jnp.float32)
        y2 = y2_ref[:, j * _KC:(j + 1) * _KC]
        score = (x2 + y2) - 2.0 * dots
        m = jnp.min(score, axis=1, keepdims=True)
        iota = jax.lax.broadcasted_iota(jnp.int32, score.shape, 1)
        arg = jnp.min(jnp.where(score == m, iota, _KC),
                      axis=1, keepdims=True) + j * _KC
        if best_v is None:
            best_v, best_i = m, arg
        else:
            take = m < best_v
            best_v = jnp.where(take, m, best_v)
            best_i = jnp.where(take, arg, best_i)

    col = jax.lax.broadcasted_iota(jnp.int32, o_ref.shape, 1)
    o_ref[...] = jnp.where(col == i % grp, best_i, o_ref[...])


def kernel(embeddings, table):
    B, S, D = embeddings.shape
    K = table.shape[0]
    N = B * S
    nk = K // _KC
    grp = min(_GRP, N // _TN)
    x = embeddings.reshape(N, D)
    tt = table.T
    out = pl.pallas_call(
        functools.partial(_vq_body, nk=nk, grp=grp),
        grid=(N // _TN,),
        in_specs=[
            pl.BlockSpec((_TN, D), lambda i: (i, 0)),
            pl.BlockSpec((D, K), lambda i: (0, 0)),
        ],
        out_specs=pl.BlockSpec((_TN, grp), lambda i: (0, i // grp)),
        out_shape=jax.ShapeDtypeStruct((_TN, N // _TN), jnp.int32),
        scratch_shapes=[pltpu.VMEM((1, K), jnp.float32)],
    )(x, tt)
    return out.T.reshape(B, S)


# speedup vs baseline: 2.2847x; 1.0483x over previous
import functools
import jax
import jax.numpy as jnp
from jax.experimental import pallas as pl
from jax.experimental.pallas import tpu as pltpu

_TN = 512
_KC = 1024
_GRP = 128


def _vq_body(x_ref, tt_ref, o_ref, y2_ref, *, nk, grp):
    i = pl.program_id(0)

    @pl.when(i == 0)
    def _norms():
        tt = tt_ref[...]
        y2_ref[...] = jnp.sum(tt * tt, axis=0, keepdims=True)

    x = x_ref[...]
    x2 = jnp.sum(x * x, axis=1, keepdims=True)

    best_v = None
    best_i = None
    for j in range(nk):
        tt = tt_ref[:, j * _KC:(j + 1) * _KC]
        dots = jax.lax.dot_general(
            x, tt, (((1,), (0,)), ((), ())),
            preferred_element_type=jnp.float32)
        y2 = y2_ref[:, j * _KC:(j + 1) * _KC]
        score = (x2 + y2) - 2.0 * dots
        m = jnp.min(score, axis=1, keepdims=True)
        iota = jax.lax.broadcasted_iota(jnp.int32, score.shape, 1)
        arg = jnp.min(jnp.where(score == m, iota, _KC),
                      axis=1, keepdims=True) + j * _KC
        if best_v is None:
            best_v, best_i = m, arg
        else:
            take = m < best_v
            best_v = jnp.where(take, m, best_v)
            best_i = jnp.where(take, arg, best_i)

    col = jax.lax.broadcasted_iota(jnp.int32, o_ref.shape, 1)
    o_ref[...] = jnp.where(col == i % grp, best_i, o_ref[...])


def kernel(embeddings, table):
    B, S, D = embeddings.shape
    K = table.shape[0]
    N = B * S
    nk = K // _KC
    grp = min(_GRP, N // _TN)
    x = embeddings.reshape(N, D)
    tt = table.T
    out = pl.pallas_call(
        functools.partial(_vq_body, nk=nk, grp=grp),
        grid=(N // _TN,),
        in_specs=[
            pl.BlockSpec((_TN, D), lambda i: (i, 0)),
            pl.BlockSpec((D, K), lambda i: (0, 0)),
        ],
        out_specs=pl.BlockSpec((_TN, grp), lambda i: (0, i // grp)),
        out_shape=jax.ShapeDtypeStruct((_TN, N // _TN), jnp.int32),
        scratch_shapes=[pltpu.VMEM((1, K), jnp.float32)],
    )(x, tt)
    return out.T.reshape(B, S)


# speedup vs baseline: 2.4481x; 1.0715x over previous
import functools
import jax
import jax.numpy as jnp
from jax.experimental import pallas as pl
from jax.experimental.pallas import tpu as pltpu

_TN = 1024
_KC = 1024
_GRP = 128


def _vq_body(x_ref, tt_ref, o_ref, y2_ref, *, nk, grp):
    i = pl.program_id(0)

    @pl.when(i == 0)
    def _norms():
        tt = tt_ref[...]
        y2_ref[...] = jnp.sum(tt * tt, axis=0, keepdims=True)

    x = x_ref[...]
    x2 = jnp.sum(x * x, axis=1, keepdims=True)

    best_v = None
    best_i = None
    for j in range(nk):
        tt = tt_ref[:, j * _KC:(j + 1) * _KC]
        dots = jax.lax.dot_general(
            x, tt, (((1,), (0,)), ((), ())),
            preferred_element_type=jnp.float32)
        y2 = y2_ref[:, j * _KC:(j + 1) * _KC]
        score = (x2 + y2) - 2.0 * dots
        m = jnp.min(score, axis=1, keepdims=True)
        iota = jax.lax.broadcasted_iota(jnp.int32, score.shape, 1)
        arg = jnp.min(jnp.where(score == m, iota, _KC),
                      axis=1, keepdims=True) + j * _KC
        if best_v is None:
            best_v, best_i = m, arg
        else:
            take = m < best_v
            best_v = jnp.where(take, m, best_v)
            best_i = jnp.where(take, arg, best_i)

    col = jax.lax.broadcasted_iota(jnp.int32, o_ref.shape, 1)
    o_ref[...] = jnp.where(col == i % grp, best_i, o_ref[...])


def kernel(embeddings, table):
    B, S, D = embeddings.shape
    K = table.shape[0]
    N = B * S
    nk = K // _KC
    grp = min(_GRP, N // _TN)
    x = embeddings.reshape(N, D)
    tt = table.T
    out = pl.pallas_call(
        functools.partial(_vq_body, nk=nk, grp=grp),
        grid=(N // _TN,),
        in_specs=[
            pl.BlockSpec((_TN, D), lambda i: (i, 0)),
            pl.BlockSpec((D, K), lambda i: (0, 0)),
        ],
        out_specs=pl.BlockSpec((_TN, grp), lambda i: (0, i // grp)),
        out_shape=jax.ShapeDtypeStruct((_TN, N // _TN), jnp.int32),
        scratch_shapes=[pltpu.VMEM((1, K), jnp.float32)],
    )(x, tt)
    return out.T.reshape(B, S)


# speedup vs baseline: 2.6850x; 1.0968x over previous
import functools
import jax
import jax.numpy as jnp
from jax.experimental import pallas as pl
from jax.experimental.pallas import tpu as pltpu

_TN = 2048
_KC = 1024
_GRP = 128


def _vq_body(x_ref, tt_ref, o_ref, y2_ref, *, nk, grp):
    i = pl.program_id(0)

    @pl.when(i == 0)
    def _norms():
        tt = tt_ref[...]
        y2_ref[...] = jnp.sum(tt * tt, axis=0, keepdims=True)

    x = x_ref[...]
    x2 = jnp.sum(x * x, axis=1, keepdims=True)

    best_v = None
    best_i = None
    for j in range(nk):
        tt = tt_ref[:, j * _KC:(j + 1) * _KC]
        dots = jax.lax.dot_general(
            x, tt, (((1,), (0,)), ((), ())),
            preferred_element_type=jnp.float32)
        y2 = y2_ref[:, j * _KC:(j + 1) * _KC]
        score = (x2 + y2) - 2.0 * dots
        m = jnp.min(score, axis=1, keepdims=True)
        iota = jax.lax.broadcasted_iota(jnp.int32, score.shape, 1)
        arg = jnp.min(jnp.where(score == m, iota, _KC),
                      axis=1, keepdims=True) + j * _KC
        if best_v is None:
            best_v, best_i = m, arg
        else:
            take = m < best_v
            best_v = jnp.where(take, m, best_v)
            best_i = jnp.where(take, arg, best_i)

    col = jax.lax.broadcasted_iota(jnp.int32, o_ref.shape, 1)
    o_ref[...] = jnp.where(col == i % grp, best_i, o_ref[...])


def kernel(embeddings, table):
    B, S, D = embeddings.shape
    K = table.shape[0]
    N = B * S
    nk = K // _KC
    grp = min(_GRP, N // _TN)
    x = embeddings.reshape(N, D)
    tt = table.T
    out = pl.pallas_call(
        functools.partial(_vq_body, nk=nk, grp=grp),
        grid=(N // _TN,),
        in_specs=[
            pl.BlockSpec((_TN, D), lambda i: (i, 0)),
            pl.BlockSpec((D, K), lambda i: (0, 0)),
        ],
        out_specs=pl.BlockSpec((_TN, grp), lambda i: (0, i // grp)),
        out_shape=jax.ShapeDtypeStruct((_TN, N // _TN), jnp.int32),
        scratch_shapes=[pltpu.VMEM((1, K), jnp.float32)],
    )(x, tt)
    return out.T.reshape(B, S)
